# baseline (speedup 1.0000x reference)
import functools
import jax
import jax.numpy as jnp
from jax import lax
from jax.experimental import pallas as pl
from jax.experimental.pallas import tpu as pltpu
from jax.experimental.pallas import tpu_sc as plsc

_B, _N, _K, _FB = 76, 151, 10, 5
_H, _DEPTH = 128, 3
_AF = 89
_NP = 160
_FBP = 16
_NW = 32
_MPW = 3
_R = _B * _NP
_BLK = 1216
_CHR = 8
_CHS = _CHR * _K
_NCH = _NP // _CHR


def _gather_mol(A_hbm, gidx_hbm, z_hbm, S_hbm,
                rows_v, S_sh, gidx_v, dst_v, sem, sid, b, bond):
    pltpu.sync_copy(gidx_hbm.at[b], gidx_v)
    pltpu.sync_copy(z_hbm, S_sh.at[pl.ds(sid * _NP, _NP)])
    if bond is not None:
        bond_hbm, bgidx_hbm, zb_hbm, Sb_hbm, brows_v, Sb_sh, bgidx_v, semb \
            = bond
        pltpu.sync_copy(bgidx_hbm.at[b], bgidx_v)
        pltpu.sync_copy(zb_hbm, Sb_sh.at[pl.ds(sid * _NP, _NP)])

    def cbody(c, _):
        atom_dma = pltpu.async_copy(A_hbm.at[gidx_v.at[c]], rows_v, sem)
        if bond is not None:
            bond_dma = pltpu.async_copy(bond_hbm.at[bgidx_v.at[c]],
                                        brows_v, semb)
        atom_dma.wait()
        pltpu.sync_copy(rows_v, S_sh.at[dst_v.at[c]], add=True)
        if bond is not None:
            bond_dma.wait()
            pltpu.sync_copy(brows_v, Sb_sh.at[dst_v.at[c]], add=True)
        return 0

    lax.fori_loop(0, _NCH, cbody, 0)
    pltpu.sync_copy(S_sh.at[pl.ds(sid * _NP, _NP)],
                    S_hbm.at[pl.ds(b * _NP, _NP)])
    if bond is not None:
        pltpu.sync_copy(Sb_sh.at[pl.ds(sid * _NP, _NP)],
                        Sb_hbm.at[pl.ds(b * _NP, _NP)])


def _sc_body_atom(A_hbm, gidx_hbm, dst_hbm, z_hbm, S_hbm,
                  rows_v, S_sh, gidx_v, dst_v, sem):
    sid = lax.axis_index("s")
    wid = sid * 2 + lax.axis_index("c")
    pltpu.sync_copy(dst_hbm.at[sid], dst_v)
    for m in range(_MPW):
        b = wid + _NW * m

        @pl.when(b < _B)
        def _():
            _gather_mol(A_hbm, gidx_hbm, z_hbm, S_hbm,
                        rows_v, S_sh, gidx_v, dst_v, sem, sid, b, None)


def _sc_body_atom_bond(A_hbm, bond_hbm, gidx_hbm, bgidx_hbm, dst_hbm,
                       z_hbm, zb_hbm, S_hbm, Sb_hbm,
                       rows_v, S_sh, gidx_v, dst_v, sem,
                       brows_v, Sb_sh, bgidx_v, semb):
    sid = lax.axis_index("s")
    wid = sid * 2 + lax.axis_index("c")
    pltpu.sync_copy(dst_hbm.at[sid], dst_v)
    for m in range(_MPW):
        b = wid + _NW * m

        @pl.when(b < _B)
        def _():
            _gather_mol(A_hbm, gidx_hbm, z_hbm, S_hbm,
                        rows_v, S_sh, gidx_v, dst_v, sem, sid, b,
                        (bond_hbm, bgidx_hbm, zb_hbm, Sb_hbm,
                         brows_v, Sb_sh, bgidx_v, semb))


_mesh = plsc.VectorSubcoreMesh(core_axis_name="c", subcore_axis_name="s")
_sc_params = pltpu.CompilerParams(needs_layout_passes=False,
                                  disable_bounds_checks=True)

_sc_gather = pl.kernel(
    _sc_body_atom,
    mesh=_mesh,
    compiler_params=_sc_params,
    out_type=jax.ShapeDtypeStruct((_R, _H), jnp.float32),
    scratch_types=[
        pltpu.VMEM((_CHS, _H), jnp.float32),
        pltpu.VMEM_SHARED((16 * _NP, _H), jnp.float32),
        pltpu.VMEM((_NCH, _CHS), jnp.int32),
        pltpu.VMEM((_NCH, _CHS), jnp.int32),
        pltpu.SemaphoreType.DMA,
    ],
)

_sc_gather_ab = pl.kernel(
    _sc_body_atom_bond,
    mesh=_mesh,
    compiler_params=_sc_params,
    out_type=(jax.ShapeDtypeStruct((_R, _H), jnp.float32),
              jax.ShapeDtypeStruct((_R, _H), jnp.float32)),
    scratch_types=[
        pltpu.VMEM((_CHS, _H), jnp.float32),
        pltpu.VMEM_SHARED((16 * _NP, _H), jnp.float32),
        pltpu.VMEM((_NCH, _CHS), jnp.int32),
        pltpu.VMEM((_NCH, _CHS), jnp.int32),
        pltpu.SemaphoreType.DMA,
        pltpu.VMEM((_CHS, _H), jnp.float32),
        pltpu.VMEM_SHARED((16 * _NP, _H), jnp.float32),
        pltpu.VMEM((_NCH, _CHS), jnp.int32),
        pltpu.SemaphoreType.DMA,
    ],
)


def _tc_pre_body(x_ref, WaT_ref, out_ref):
    out_ref[...] = jnp.dot(x_ref[...], WaT_ref[...],
                           preferred_element_type=jnp.float32)


def _tc_step_body(a_ref, s_ref, bsum_ref, W2aT_ref, W2bT_ref, W1aT_ref,
                  W1bT_ref, bU1_ref, out_ref):
    f32 = jnp.float32
    pid = pl.program_id(0)
    nei = (jnp.dot(s_ref[...], W2aT_ref[...], preferred_element_type=f32)
           + jnp.dot(bsum_ref[...], W2bT_ref[...], preferred_element_type=f32))
    a_new = (jnp.dot(a_ref[...], W1aT_ref[...], preferred_element_type=f32)
             + jnp.dot(nei, W1bT_ref[...], preferred_element_type=f32)
             + bU1_ref[...])
    row = lax.broadcasted_iota(jnp.int32, (_BLK, _H), 0) + pid * _BLK
    valid = (lax.rem(row, _NP) < _N).astype(f32)
    out_ref[...] = a_new * valid


def _row_spec():
    return pl.BlockSpec((_BLK, _H), lambda i: (i, 0))


_tc_pre = functools.partial(
    pl.pallas_call, _tc_pre_body,
    grid=(_R // _BLK,),
    in_specs=[_row_spec(), pl.BlockSpec((_H, _H), lambda i: (0, 0))],
    out_specs=_row_spec(),
    out_shape=jax.ShapeDtypeStruct((_R, _H), jnp.float32),
)()

_tc_step = functools.partial(
    pl.pallas_call, _tc_step_body,
    grid=(_R // _BLK,),
    in_specs=[
        _row_spec(), _row_spec(),
        pl.BlockSpec((_BLK, _H), lambda i: (i, 0)),
        pl.BlockSpec((_H, _H), lambda i: (0, 0)),
        pl.BlockSpec((_H, _H), lambda i: (0, 0)),
        pl.BlockSpec((_H, _H), lambda i: (0, 0)),
        pl.BlockSpec((_H, _H), lambda i: (0, 0)),
        pl.BlockSpec((1, _H), lambda i: (0, 0)),
    ],
    out_specs=_row_spec(),
    out_shape=jax.ShapeDtypeStruct((_R, _H), jnp.float32),
)()


def kernel(input_atom, input_bond, atom_nei_idx, bond_nei_idx, num_nbs,
           W_atom, W_U2, b_U2, W_U1, b_U1):
    f32 = jnp.float32
    xp = jnp.zeros((_B, _NP, _H), f32)
    xp = xp.at[:, :_N, :_AF].set(input_atom)
    bond_tbl = jnp.zeros((_B, _NP, _H), f32)
    bond_tbl = bond_tbl.at[:, :_N, :_FB].set(input_bond)
    bond_tbl = bond_tbl.at[:, :_N, _FB].set(1.0)

    mask = jnp.arange(_K, dtype=jnp.int32)[None, None, :] \
        < jnp.clip(num_nbs, 0, _K)[:, :, None]
    base = (jnp.arange(_B, dtype=jnp.int32) * _NP)[:, None, None]
    pad_row = base + (_NP - 1)
    ga = jnp.where(mask, atom_nei_idx + base, pad_row)
    gb = jnp.where(mask, bond_nei_idx[..., 0] + base, pad_row)
    gpad = jnp.broadcast_to(pad_row, (_B, _NP - _N, _K))
    gidx_a = jnp.concatenate([ga, gpad], 1).reshape(_B, _NCH, _CHS)
    gidx_b = jnp.concatenate([gb, gpad], 1).reshape(_B, _NCH, _CHS)
    dst = (jnp.arange(16, dtype=jnp.int32)[:, None, None] * _NP
           + jnp.arange(_NCH, dtype=jnp.int32)[None, :, None] * _CHR
           + jnp.arange(_CHS, dtype=jnp.int32)[None, None, :] // _K)

    zH = jnp.zeros((_NP, _H), f32)

    WaT = jnp.zeros((_H, _H), f32).at[:_AF, :].set(W_atom.T)
    W2aT = W_U2[:, :_H].T
    W2b_aug = jnp.concatenate([W_U2[:, _H:], b_U2[:, None]], axis=1)
    W2bT = jnp.zeros((_H, _H), f32).at[:_FB + 1, :].set(W2b_aug.T)
    W1aT = W_U1[:, :_H].T
    W1bT = W_U1[:, _H:].T
    bU1 = b_U1[None, :]

    A = _tc_pre(xp.reshape(_R, _H), WaT)
    S, bsum = _sc_gather_ab(A, bond_tbl.reshape(_R, _H),
                            gidx_a, gidx_b, dst, zH, zH)
    A = _tc_step(A, S, bsum, W2aT, W2bT, W1aT, W1bT, bU1)
    for _ in range(_DEPTH - 1):
        S = _sc_gather(A, gidx_a, dst, zH)
        A = _tc_step(A, S, bsum, W2aT, W2bT, W1aT, W1bT, bU1)
    return A.reshape(_B, _NP, _H)[:, :_N, :]
